# baseline (speedup 1.0000x reference)
import functools

import jax
import jax.numpy as jnp
from jax.experimental import pallas as pl
from jax.experimental.pallas import tpu as pltpu

_G = 128


def _gru_kernel(tok_ref, embt_ref, wi_f_ref, wh_f_ref, bi_f_ref, bh_f_ref,
                wi_b_ref, wh_b_ref, bi_b_ref, bh_b_ref, out_ref, *, T, H, V):
    blk = tok_ref.shape[1]
    tok = tok_ref[...]
    lengths = jnp.sum((tok != 0).astype(jnp.int32), axis=0, keepdims=True)
    embt = embt_ref[...]

    def make_tab(wi_ref, bi_ref, bh_ref):
        tab = jnp.dot(wi_ref[...], embt,
                      preferred_element_type=jnp.float32) + bi_ref[...]
        bh = bh_ref[...]
        rz = jnp.concatenate(
            [tab[:2 * _G] + bh[:2 * _G], tab[2 * _G:]], axis=0)
        bhn = jnp.broadcast_to(bh[2 * _G:], (_G, blk))
        return rz, bhn

    tab_f, bhn_f = make_tab(wi_f_ref, bi_f_ref, bh_f_ref)
    tab_b, bhn_b = make_tab(wi_b_ref, bi_b_ref, bh_b_ref)
    wh_f = wh_f_ref[...]
    wh_b = wh_b_ref[...]

    def step(h, k, tab, wh, bhn):
        idx = jnp.broadcast_to(tok[k:k + 1, :], (3 * _G, blk))
        gx = jnp.take_along_axis(tab, idx, axis=1)
        gh = jnp.dot(wh, h, preferred_element_type=jnp.float32)
        r = jax.nn.sigmoid(gx[:_G] + gh[:_G])
        z = jax.nn.sigmoid(gx[_G:2 * _G] + gh[_G:2 * _G])
        n = jnp.tanh(gx[2 * _G:] + r * (gh[2 * _G:] + bhn))
        h_new = (1.0 - z) * n + z * h
        return jnp.where(k < lengths, h_new, h)

    hf = jnp.zeros((_G, blk), jnp.float32)
    hb = jnp.zeros((_G, blk), jnp.float32)
    for k in range(T):
        hf = step(hf, k, tab_f, wh_f, bhn_f)
        hb = step(hb, T - 1 - k, tab_b, wh_b, bhn_b)
    hft = hf.T
    hbt = hb.T
    out_ref[...] = jnp.concatenate([hft[:, :H], hbt[:, :H]], axis=1)


def _pack_w(W, H):
    K = W.shape[1]
    return jnp.pad(W.reshape(3, H, K), ((0, 0), (0, _G - H), (0, 0))
                   ).reshape(3 * _G, K)


def _pack_b(b, H):
    return jnp.pad(b.reshape(3, H), ((0, 0), (0, _G - H))).reshape(3 * _G, 1)


def kernel(c, q, ch, qh, char_emb, Wi_f, Wh_f, bi_f, bh_f,
           Wi_b, Wh_b, bi_b, bh_b):
    T = ch.shape[2]
    N1 = ch.shape[0] * ch.shape[1]
    N2 = qh.shape[0] * qh.shape[1]
    H = Wh_f.shape[1]
    V = char_emb.shape[0]
    tokens = jnp.concatenate(
        [ch.reshape(N1, T), qh.reshape(N2, T)], axis=0).astype(jnp.int32)
    N = N1 + N2

    blk = 768
    npad = (-N) % blk
    if npad:
        tokens = jnp.pad(tokens, ((0, npad), (0, 0)))
    ntot = N + npad
    tokt = tokens.T

    embt = char_emb.T
    wi_f = _pack_w(Wi_f, H)
    wi_b = _pack_w(Wi_b, H)
    wh_f = jnp.pad(_pack_w(Wh_f, H), ((0, 0), (0, _G - H)))
    wh_b = jnp.pad(_pack_w(Wh_b, H), ((0, 0), (0, _G - H)))
    pbi_f = _pack_b(bi_f, H)
    pbi_b = _pack_b(bi_b, H)
    pbh_f = _pack_b(bh_f, H)
    pbh_b = _pack_b(bh_b, H)

    full = lambda a: pl.BlockSpec(a.shape, lambda i: (0,) * a.ndim)
    out = pl.pallas_call(
        functools.partial(_gru_kernel, T=T, H=H, V=V),
        grid=(ntot // blk,),
        in_specs=[
            pl.BlockSpec((T, blk), lambda i: (0, i)),
            full(embt), full(wi_f), full(wh_f), full(pbi_f),
            full(pbh_f), full(wi_b), full(wh_b), full(pbi_b), full(pbh_b),
        ],
        out_specs=pl.BlockSpec((blk, 2 * H), lambda i: (i, 0)),
        out_shape=jax.ShapeDtypeStruct((ntot, 2 * H), jnp.float32),
        compiler_params=pltpu.CompilerParams(
            dimension_semantics=("parallel",)),
    )(tokt, embt, wi_f, wh_f, pbi_f, pbh_f,
      wi_b, wh_b, pbi_b, pbh_b)
    return out[:N1], out[N1:N]


# speedup vs baseline: 2.1747x; 2.1747x over previous
import functools

import jax
import jax.numpy as jnp
from jax.experimental import pallas as pl
from jax.experimental.pallas import tpu as pltpu

_G = 128


def _gru_kernel(tok_ref, emb_ref, wit_f_ref, wht_f_ref, bi_f_ref, bh_f_ref,
                wit_b_ref, wht_b_ref, bi_b_ref, bh_b_ref, out_ref, *, T, H, V):
    blk = tok_ref.shape[0]
    tok = tok_ref[...]
    lengths = jnp.sum((tok != 0).astype(jnp.int32), axis=1, keepdims=True)
    emb = emb_ref[...]
    iota = jax.lax.broadcasted_iota(jnp.int32, (blk, V), 1)

    def make_tab(wit_ref, bi_ref):
        return (jnp.dot(emb, wit_ref[...],
                        preferred_element_type=jnp.float32)
                + bi_ref[...]).astype(jnp.bfloat16)

    tab_f = make_tab(wit_f_ref, bi_f_ref)
    tab_b = make_tab(wit_b_ref, bi_b_ref)
    wht_f = wht_f_ref[...].astype(jnp.bfloat16)
    wht_b = wht_b_ref[...].astype(jnp.bfloat16)
    bh_f = bh_f_ref[...]
    bh_b = bh_b_ref[...]

    def step(h, k, tab, wht, bh):
        oh = (tok[:, k:k + 1] == iota).astype(jnp.bfloat16)
        gx = jnp.dot(oh, tab, preferred_element_type=jnp.float32)
        gh = jnp.dot(h.astype(jnp.bfloat16), wht,
                     preferred_element_type=jnp.float32) + bh
        r = jax.nn.sigmoid(gx[:, :_G] + gh[:, :_G])
        z = jax.nn.sigmoid(gx[:, _G:2 * _G] + gh[:, _G:2 * _G])
        n = jnp.tanh(gx[:, 2 * _G:] + r * gh[:, 2 * _G:])
        h_new = (1.0 - z) * n + z * h
        return jnp.where(k < lengths, h_new, h)

    hf = jnp.zeros((blk, _G), jnp.float32)
    hb = jnp.zeros((blk, _G), jnp.float32)
    for k in range(T):
        hf = step(hf, k, tab_f, wht_f, bh_f)
        hb = step(hb, T - 1 - k, tab_b, wht_b, bh_b)
    out_ref[...] = jnp.concatenate([hf[:, :H], hb[:, :H]], axis=1)


def _pack_w(W, H):
    K = W.shape[1]
    W3 = jnp.pad(W.reshape(3, H, K), ((0, 0), (0, _G - H), (0, 0)))
    return W3.reshape(3 * _G, K).T


def _pack_b(b, H):
    return jnp.pad(b.reshape(3, H), ((0, 0), (0, _G - H))).reshape(1, 3 * _G)


def kernel(c, q, ch, qh, char_emb, Wi_f, Wh_f, bi_f, bh_f,
           Wi_b, Wh_b, bi_b, bh_b):
    T = ch.shape[2]
    N1 = ch.shape[0] * ch.shape[1]
    N2 = qh.shape[0] * qh.shape[1]
    H = Wh_f.shape[1]
    V = char_emb.shape[0]
    tokens = jnp.concatenate(
        [ch.reshape(N1, T), qh.reshape(N2, T)], axis=0).astype(jnp.int32)
    N = N1 + N2

    blk = 800
    npad = (-N) % blk
    if npad:
        tokens = jnp.pad(tokens, ((0, npad), (0, 0)))
    ntot = N + npad

    wit_f = _pack_w(Wi_f, H)
    wit_b = _pack_w(Wi_b, H)
    wht_f = jnp.pad(_pack_w(Wh_f, H), ((0, _G - H), (0, 0)))
    wht_b = jnp.pad(_pack_w(Wh_b, H), ((0, _G - H), (0, 0)))
    pbi_f = _pack_b(bi_f, H)
    pbi_b = _pack_b(bi_b, H)
    pbh_f = _pack_b(bh_f, H)
    pbh_b = _pack_b(bh_b, H)

    full = lambda a: pl.BlockSpec(a.shape, lambda i: (0,) * a.ndim)
    out = pl.pallas_call(
        functools.partial(_gru_kernel, T=T, H=H, V=V),
        grid=(ntot // blk,),
        in_specs=[
            pl.BlockSpec((blk, T), lambda i: (i, 0)),
            full(char_emb), full(wit_f), full(wht_f), full(pbi_f),
            full(pbh_f), full(wit_b), full(wht_b), full(pbi_b), full(pbh_b),
        ],
        out_specs=pl.BlockSpec((blk, 2 * H), lambda i: (i, 0)),
        out_shape=jax.ShapeDtypeStruct((ntot, 2 * H), jnp.float32),
        compiler_params=pltpu.CompilerParams(
            dimension_semantics=("parallel",)),
    )(tokens, char_emb, wit_f, wht_f, pbi_f, pbh_f,
      wit_b, wht_b, pbi_b, pbh_b)
    return out[:N1], out[N1:N]


# speedup vs baseline: 2.2068x; 1.0147x over previous
import functools

import jax
import jax.numpy as jnp
from jax.experimental import pallas as pl
from jax.experimental.pallas import tpu as pltpu

_G = 128


def _gru_kernel(tok_ref, emb_ref, wit_f_ref, wht_f_ref, bi_f_ref, bh_f_ref,
                wit_b_ref, wht_b_ref, bi_b_ref, bh_b_ref, out_ref, *, T, H, V):
    blk = tok_ref.shape[0]
    tok = tok_ref[...]
    lengths = jnp.sum((tok != 0).astype(jnp.int32), axis=1, keepdims=True)
    emb = emb_ref[...]
    iota = jax.lax.broadcasted_iota(jnp.int32, (blk, V), 1)

    def make_tab(wit_ref, bi_ref):
        return (jnp.dot(emb, wit_ref[...],
                        preferred_element_type=jnp.float32)
                + bi_ref[...]).astype(jnp.bfloat16)

    tab_f = make_tab(wit_f_ref, bi_f_ref)
    tab_b = make_tab(wit_b_ref, bi_b_ref)
    wht_f = wht_f_ref[...].astype(jnp.bfloat16)
    wht_b = wht_b_ref[...].astype(jnp.bfloat16)
    bh_f = bh_f_ref[...]
    bh_b = bh_b_ref[...]

    def step(h, k, tab, wht, bh):
        oh = (tok[:, k:k + 1] == iota).astype(jnp.bfloat16)
        gx = jnp.dot(oh, tab, preferred_element_type=jnp.float32)
        gh = jnp.dot(h.astype(jnp.bfloat16), wht,
                     preferred_element_type=jnp.float32) + bh
        r = 0.5 * jnp.tanh(0.5 * (gx[:, :_G] + gh[:, :_G])) + 0.5
        z = 0.5 * jnp.tanh(0.5 * (gx[:, _G:2 * _G] + gh[:, _G:2 * _G])) + 0.5
        n = jnp.tanh(gx[:, 2 * _G:] + r * gh[:, 2 * _G:])
        h_new = (1.0 - z) * n + z * h
        return jnp.where(k < lengths, h_new, h)

    hf = jnp.zeros((blk, _G), jnp.float32)
    hb = jnp.zeros((blk, _G), jnp.float32)
    for k in range(T):
        hf = step(hf, k, tab_f, wht_f, bh_f)
        hb = step(hb, T - 1 - k, tab_b, wht_b, bh_b)
    out_ref[...] = jnp.concatenate([hf[:, :H], hb[:, :H]], axis=1)


def _pack_w(W, H):
    K = W.shape[1]
    W3 = jnp.pad(W.reshape(3, H, K), ((0, 0), (0, _G - H), (0, 0)))
    return W3.reshape(3 * _G, K).T


def _pack_b(b, H):
    return jnp.pad(b.reshape(3, H), ((0, 0), (0, _G - H))).reshape(1, 3 * _G)


def kernel(c, q, ch, qh, char_emb, Wi_f, Wh_f, bi_f, bh_f,
           Wi_b, Wh_b, bi_b, bh_b):
    T = ch.shape[2]
    N1 = ch.shape[0] * ch.shape[1]
    N2 = qh.shape[0] * qh.shape[1]
    H = Wh_f.shape[1]
    V = char_emb.shape[0]
    tokens = jnp.concatenate(
        [ch.reshape(N1, T), qh.reshape(N2, T)], axis=0).astype(jnp.int32)
    N = N1 + N2

    blk = 800
    npad = (-N) % blk
    if npad:
        tokens = jnp.pad(tokens, ((0, npad), (0, 0)))
    ntot = N + npad

    wit_f = _pack_w(Wi_f, H)
    wit_b = _pack_w(Wi_b, H)
    wht_f = jnp.pad(_pack_w(Wh_f, H), ((0, _G - H), (0, 0)))
    wht_b = jnp.pad(_pack_w(Wh_b, H), ((0, _G - H), (0, 0)))
    pbi_f = _pack_b(bi_f, H)
    pbi_b = _pack_b(bi_b, H)
    pbh_f = _pack_b(bh_f, H)
    pbh_b = _pack_b(bh_b, H)

    full = lambda a: pl.BlockSpec(a.shape, lambda i: (0,) * a.ndim)
    out = pl.pallas_call(
        functools.partial(_gru_kernel, T=T, H=H, V=V),
        grid=(ntot // blk,),
        in_specs=[
            pl.BlockSpec((blk, T), lambda i: (i, 0)),
            full(char_emb), full(wit_f), full(wht_f), full(pbi_f),
            full(pbh_f), full(wit_b), full(wht_b), full(pbi_b), full(pbh_b),
        ],
        out_specs=pl.BlockSpec((blk, 2 * H), lambda i: (i, 0)),
        out_shape=jax.ShapeDtypeStruct((ntot, 2 * H), jnp.float32),
        compiler_params=pltpu.CompilerParams(
            dimension_semantics=("parallel",)),
    )(tokens, char_emb, wit_f, wht_f, pbi_f, pbh_f,
      wit_b, wht_b, pbi_b, pbh_b)
    return out[:N1], out[N1:N]
